# baseline (speedup 1.0000x reference)
import functools

import jax
import jax.numpy as jnp
from jax import lax
from jax.experimental import pallas as pl
from jax.experimental.pallas import tpu as pltpu
import jax.experimental.pallas.tpu_sc as plsc

N = 10000
F = 128
T = 10
NP = 10240
NW = 32
CPW = NP // NW
K = 8
NCH = CPW // K
BI = 400


def _gcn_body(x_ref, w_ref, b_ref, adj_ref, h_ref, xw_scr):
    @pl.when(pl.program_id(0) == 0)
    def _():
        xw_scr[...] = jnp.dot(x_ref[...], w_ref[...],
                              preferred_element_type=jnp.float32)

    z = jnp.dot(adj_ref[...], xw_scr[...],
                preferred_element_type=jnp.float32) + b_ref[...]
    h_ref[...] = jnp.where(z > 0, z, jnp.exp(jnp.minimum(z, 0.0)) - 1.0)


def _gcn(x, w, b, adj):
    return pl.pallas_call(
        _gcn_body,
        grid=(N // BI,),
        in_specs=[
            pl.BlockSpec((N, F), lambda i: (0, 0)),
            pl.BlockSpec((F, F), lambda i: (0, 0)),
            pl.BlockSpec((1, F), lambda i: (0, 0)),
            pl.BlockSpec((BI, N), lambda i: (i, 0)),
        ],
        out_specs=pl.BlockSpec((BI, F), lambda i: (i, 0)),
        out_shape=jax.ShapeDtypeStruct((N, F), jnp.float32),
        scratch_shapes=[pltpu.VMEM((N, F), jnp.float32)],
    )(x, w, b, adj)


def _bn_body(h_ref, gam_ref, bet_ref, out_ref):
    h = h_ref[...]
    mean = jnp.mean(h, axis=0, keepdims=True)
    c = h - mean
    var = jnp.mean(c * c, axis=0, keepdims=True)
    y = c * lax.rsqrt(var + 1e-5) * gam_ref[...] + bet_ref[...]
    rn = jnp.sqrt(jnp.sum(y * y, axis=1, keepdims=True))
    g = y / jnp.maximum(rn, 1e-8)
    out_ref[0:N, :] = g
    out_ref[N:NP, :] = jnp.zeros((NP - N, F), jnp.float32)


def _bn_normalize(h, gam, bet):
    return pl.pallas_call(
        _bn_body,
        out_shape=jax.ShapeDtypeStruct((NP, F), jnp.float32),
    )(h, gam, bet)


def _sc_sims_body(g_hbm, idx2_hbm, out_hbm, qall, idxall, tbufA, tbufB,
                  accall, semA, semB):
    wid = lax.axis_index("s") * 2 + lax.axis_index("c")
    base = wid * CPW

    pltpu.sync_copy(idx2_hbm.at[pl.ds(wid * NCH, NCH)], idxall)
    pltpu.sync_copy(g_hbm.at[pl.ds(base, CPW)], qall)
    tbufs = [tbufA, tbufB]
    sems = [semA, semB]
    NB = 2
    for b in range(NB):
        pltpu.async_copy(g_hbm.at[idxall.at[b]], tbufs[b], sems[b])

    def compute_chunk(ch, tbuf):
        def i_body(i, c2):
            ii = ch * K + i
            qs = [qall[ii, 16 * c:16 * (c + 1)] for c in range(8)]
            for t in range(T):
                r = i * T + t
                p0 = qs[0] * tbuf[r, 0:16]
                p1 = qs[1] * tbuf[r, 16:32]
                p2 = qs[2] * tbuf[r, 32:48]
                p3 = qs[3] * tbuf[r, 48:64]
                p4 = qs[4] * tbuf[r, 64:80]
                p5 = qs[5] * tbuf[r, 80:96]
                p6 = qs[6] * tbuf[r, 96:112]
                p7 = qs[7] * tbuf[r, 112:128]
                acc = ((p0 + p1) + (p2 + p3)) + ((p4 + p5) + (p6 + p7))
                o = ii * (16 * T) + 16 * t
                accall[pl.ds(o, 16)] = acc
            return c2

        lax.fori_loop(0, K, i_body, 0)

    def ring_body(j, carry):
        ch0 = NB * j
        for b in range(NB):
            ch = ch0 + b
            pltpu.make_async_copy(g_hbm.at[idxall.at[0]], tbufs[b],
                                  sems[b]).wait()
            compute_chunk(ch, tbufs[b])

            @pl.when(ch + NB < NCH)
            def _():
                pltpu.async_copy(g_hbm.at[idxall.at[ch + NB]], tbufs[b],
                                 sems[b])

        return carry

    lax.fori_loop(0, NCH // NB, ring_body, 0)
    pltpu.sync_copy(accall, out_hbm.at[pl.ds(base * 16 * T, CPW * 16 * T)])


@functools.cache
def _sc_sims():
    return pl.kernel(
        _sc_sims_body,
        out_type=jax.ShapeDtypeStruct((NP * 16 * T,), jnp.float32),
        mesh=plsc.VectorSubcoreMesh(core_axis_name="c", subcore_axis_name="s"),
        compiler_params=pltpu.CompilerParams(needs_layout_passes=False),
        scratch_types=[
            pltpu.VMEM((CPW, F), jnp.float32),
            pltpu.VMEM((NCH, K * T), jnp.int32),
            pltpu.VMEM((K * T, F), jnp.float32),
            pltpu.VMEM((K * T, F), jnp.float32),
            pltpu.VMEM((CPW * 16 * T,), jnp.float32),
            pltpu.SemaphoreType.DMA,
            pltpu.SemaphoreType.DMA,
        ],
    )


def _loss_body(acc_ref, out_ref):
    num = jnp.sum(acc_ref[:, 0:16], axis=1, keepdims=True)
    den = jnp.zeros((NP, 1), jnp.float32)
    for t in range(1, T):
        st = jnp.sum(acc_ref[:, 16 * t:16 * (t + 1)], axis=1, keepdims=True)
        den = den + jnp.exp(st)
    li = jnp.log(den) - num
    row = lax.broadcasted_iota(jnp.int32, (NP, 1), 0)
    li = jnp.where(row < N, li, 0.0)
    out_ref[...] = (jnp.sum(li) / N).reshape(1, 1)


def _loss(sims):
    return pl.pallas_call(
        _loss_body,
        out_shape=jax.ShapeDtypeStruct((1, 1), jnp.float32),
    )(sims)


def kernel(seq1, seq2, seq3, seq4, adj, aug_adj1edge, aug_adj2edge, sparse,
           msk, samp_bias1, samp_bias2, lbl, sample, W0, b0, gamma0, beta0):
    x = seq1[0]
    h = _gcn(x, W0, b0.reshape(1, F), adj)
    g = _bn_normalize(h, gamma0.reshape(1, F), beta0.reshape(1, F))
    idx2 = jnp.concatenate([
        sample.astype(jnp.int32).reshape(-1),
        jnp.zeros((NP - N) * T, jnp.int32),
    ]).reshape(NP * T // (K * T), K * T)
    accs = _sc_sims()(g, idx2).reshape(NP, 16 * T)
    return _loss(accs)[0, 0]


# speedup vs baseline: 1.2274x; 1.2274x over previous
import functools

import jax
import jax.numpy as jnp
from jax import lax
from jax.experimental import pallas as pl
from jax.experimental.pallas import tpu as pltpu
import jax.experimental.pallas.tpu_sc as plsc

N = 10000
F = 128
T = 10
NP = 10240
NW = 32
CPW = NP // NW
K = 10
NCH = CPW // K
BI = 400


def _gcn_body(x_ref, w_ref, b_ref, adj_ref, h_ref, xw_scr):
    @pl.when(pl.program_id(0) == 0)
    def _():
        xw_scr[...] = jnp.dot(x_ref[...], w_ref[...],
                              preferred_element_type=jnp.float32)

    z = jnp.dot(adj_ref[...], xw_scr[...],
                preferred_element_type=jnp.float32) + b_ref[...]
    h_ref[...] = jnp.where(z > 0, z, jnp.exp(jnp.minimum(z, 0.0)) - 1.0)


def _gcn(x, w, b, adj):
    return pl.pallas_call(
        _gcn_body,
        grid=(N // BI,),
        in_specs=[
            pl.BlockSpec((N, F), lambda i: (0, 0)),
            pl.BlockSpec((F, F), lambda i: (0, 0)),
            pl.BlockSpec((1, F), lambda i: (0, 0)),
            pl.BlockSpec((BI, N), lambda i: (i, 0)),
        ],
        out_specs=pl.BlockSpec((BI, F), lambda i: (i, 0)),
        out_shape=jax.ShapeDtypeStruct((N, F), jnp.float32),
        scratch_shapes=[pltpu.VMEM((N, F), jnp.float32)],
    )(x, w, b, adj)


def _bn_body(h_ref, gam_ref, bet_ref, out_ref):
    h = h_ref[...]
    mean = jnp.mean(h, axis=0, keepdims=True)
    c = h - mean
    var = jnp.mean(c * c, axis=0, keepdims=True)
    y = c * lax.rsqrt(var + 1e-5) * gam_ref[...] + bet_ref[...]
    rn = jnp.sqrt(jnp.sum(y * y, axis=1, keepdims=True))
    g = y / jnp.maximum(rn, 1e-8)
    out_ref[0:N, :] = g.astype(jnp.bfloat16)
    out_ref[N:NP, :] = jnp.zeros((NP - N, F), jnp.bfloat16)


def _bn_normalize(h, gam, bet):
    return pl.pallas_call(
        _bn_body,
        out_shape=jax.ShapeDtypeStruct((NP, F), jnp.bfloat16),
    )(h, gam, bet)


def _sc_sims_body(g_hbm, idx2_hbm, out_hbm, qall, idxall, tbufA, tbufB,
                  tbufC, tbufD, accall, semA, semB, semC, semD):
    wid = lax.axis_index("s") * 2 + lax.axis_index("c")
    base = wid * CPW

    pltpu.sync_copy(idx2_hbm.at[pl.ds(wid * NCH, NCH)], idxall)
    pltpu.sync_copy(g_hbm.at[pl.ds(base, CPW)], qall)
    tbufs = [tbufA, tbufB, tbufC, tbufD]
    sems = [semA, semB, semC, semD]
    NB = 4
    for b in range(NB):
        pltpu.async_copy(g_hbm.at[idxall.at[b]], tbufs[b], sems[b])

    def compute_chunk(ch, tbuf):
        def i_body(i, c2):
            ii = ch * K + i
            qs = []
            for u in range(4):
                qw = plsc.bitcast(qall[ii, 16 * u:16 * (u + 1)],
                                  jnp.bfloat16)
                qa, qb = plsc.unpack(qw, format=plsc.PackFormat.INTERLEAVED)
                qs.append(qa)
                qs.append(qb)
            for t in range(T):
                r = i * T + t
                ps = []
                for u in range(4):
                    tw = plsc.bitcast(tbuf[r, 16 * u:16 * (u + 1)],
                                      jnp.bfloat16)
                    a, b2 = plsc.unpack(tw,
                                        format=plsc.PackFormat.INTERLEAVED)
                    ps.append(qs[2 * u] * a)
                    ps.append(qs[2 * u + 1] * b2)
                acc = (((ps[0] + ps[1]) + (ps[2] + ps[3]))
                       + ((ps[4] + ps[5]) + (ps[6] + ps[7])))
                o = ii * (16 * T) + 16 * t
                accall[pl.ds(o, 16)] = acc
            return c2

        lax.fori_loop(0, K, i_body, 0)

    def ring_body(j, carry):
        ch0 = NB * j
        for b in range(NB):
            ch = ch0 + b
            pltpu.make_async_copy(g_hbm.at[idxall.at[0]], tbufs[b],
                                  sems[b]).wait()
            compute_chunk(ch, tbufs[b])

            @pl.when(ch + NB < NCH)
            def _():
                pltpu.async_copy(g_hbm.at[idxall.at[ch + NB]], tbufs[b],
                                 sems[b])

        return carry

    lax.fori_loop(0, NCH // NB, ring_body, 0)
    pltpu.sync_copy(accall, out_hbm.at[pl.ds(base * 16 * T, CPW * 16 * T)])


@functools.cache
def _sc_sims():
    return pl.kernel(
        _sc_sims_body,
        out_type=jax.ShapeDtypeStruct((NP * 16 * T,), jnp.float32),
        mesh=plsc.VectorSubcoreMesh(core_axis_name="c", subcore_axis_name="s"),
        compiler_params=pltpu.CompilerParams(needs_layout_passes=False,
                                             use_tc_tiling_on_sc=False),
        scratch_types=[
            pltpu.VMEM((CPW, F // 2), jnp.int32),
            pltpu.VMEM((NCH, K * T), jnp.int32),
            pltpu.VMEM((K * T, F // 2), jnp.int32),
            pltpu.VMEM((K * T, F // 2), jnp.int32),
            pltpu.VMEM((K * T, F // 2), jnp.int32),
            pltpu.VMEM((K * T, F // 2), jnp.int32),
            pltpu.VMEM((CPW * 16 * T,), jnp.float32),
            pltpu.SemaphoreType.DMA,
            pltpu.SemaphoreType.DMA,
            pltpu.SemaphoreType.DMA,
            pltpu.SemaphoreType.DMA,
        ],
    )


def _loss_body(acc_ref, out_ref):
    mr = lax.broadcasted_iota(jnp.int32, (16 * T, 16), 0)
    mc = lax.broadcasted_iota(jnp.int32, (16 * T, 16), 1)
    fold = jnp.where(mr // 16 == mc, 1.0, 0.0)
    s = jnp.dot(acc_ref[...], fold, preferred_element_type=jnp.float32)
    lane = lax.broadcasted_iota(jnp.int32, (NP, 16), 1)
    e = jnp.where((lane >= 1) & (lane < T), jnp.exp(s), 0.0)
    den = jnp.sum(e, axis=1, keepdims=True)
    li = jnp.log(den) - s[:, 0:1]
    row = lax.broadcasted_iota(jnp.int32, (NP, 1), 0)
    li = jnp.where(row < N, li, 0.0)
    out_ref[...] = (jnp.sum(li) / N).reshape(1, 1)


def _loss(sims):
    return pl.pallas_call(
        _loss_body,
        out_shape=jax.ShapeDtypeStruct((1, 1), jnp.float32),
    )(sims)


def kernel(seq1, seq2, seq3, seq4, adj, aug_adj1edge, aug_adj2edge, sparse,
           msk, samp_bias1, samp_bias2, lbl, sample, W0, b0, gamma0, beta0):
    x = seq1[0]
    h = _gcn(x, W0, b0.reshape(1, F), adj)
    g = _bn_normalize(h, gamma0.reshape(1, F), beta0.reshape(1, F))
    idx2 = jnp.concatenate([
        sample.astype(jnp.int32).reshape(-1),
        jnp.zeros((NP - N) * T, jnp.int32),
    ]).reshape(NP * T // (K * T), K * T)
    g_i32 = lax.bitcast_convert_type(g.reshape(NP, F // 2, 2), jnp.int32)
    accs = _sc_sims()(g_i32, idx2).reshape(NP, 16 * T)
    return _loss(accs)[0, 0]


# speedup vs baseline: 1.2734x; 1.0375x over previous
import functools

import jax
import jax.numpy as jnp
from jax import lax
from jax.experimental import pallas as pl
from jax.experimental.pallas import tpu as pltpu
import jax.experimental.pallas.tpu_sc as plsc

N = 10000
F = 128
T = 10
NP = 10240
NW = 32
CPW = NP // NW
K = 10
NCH = CPW // K
BI = 400


def _gcn_body(x_ref, w_ref, b_ref, adj_ref, h_ref, xw_scr):
    @pl.when(pl.program_id(0) == 0)
    def _():
        xw_scr[...] = jnp.dot(x_ref[...], w_ref[...],
                              preferred_element_type=jnp.float32)

    z = jnp.dot(adj_ref[...], xw_scr[...],
                preferred_element_type=jnp.float32) + b_ref[...]
    h_ref[...] = jnp.where(z > 0, z, jnp.exp(jnp.minimum(z, 0.0)) - 1.0)


def _gcn(x, w, b, adj):
    return pl.pallas_call(
        _gcn_body,
        grid=(N // BI,),
        in_specs=[
            pl.BlockSpec((N, F), lambda i: (0, 0)),
            pl.BlockSpec((F, F), lambda i: (0, 0)),
            pl.BlockSpec((1, F), lambda i: (0, 0)),
            pl.BlockSpec((BI, N), lambda i: (i, 0)),
        ],
        out_specs=pl.BlockSpec((BI, F), lambda i: (i, 0)),
        out_shape=jax.ShapeDtypeStruct((N, F), jnp.float32),
        scratch_shapes=[pltpu.VMEM((N, F), jnp.float32)],
    )(x, w, b, adj)


def _bn_body(h_ref, gam_ref, bet_ref, out_ref):
    h = h_ref[...]
    mean = jnp.mean(h, axis=0, keepdims=True)
    c = h - mean
    var = jnp.mean(c * c, axis=0, keepdims=True)
    y = c * lax.rsqrt(var + 1e-5) * gam_ref[...] + bet_ref[...]
    rn = jnp.sqrt(jnp.sum(y * y, axis=1, keepdims=True))
    g = y / jnp.maximum(rn, 1e-8)
    out_ref[0:N, :] = g.astype(jnp.bfloat16)
    out_ref[N:NP, :] = jnp.zeros((NP - N, F), jnp.bfloat16)


def _bn_normalize(h, gam, bet):
    return pl.pallas_call(
        _bn_body,
        out_shape=jax.ShapeDtypeStruct((NP, F), jnp.bfloat16),
    )(h, gam, bet)


def _sc_sims_body(g_hbm, idx2_hbm, out_hbm, qall, idxall, tbufA, tbufB,
                  tbufC, tbufD, simall, semA, semB, semC, semD):
    wid = lax.axis_index("s") * 2 + lax.axis_index("c")
    base = wid * CPW
    lane = lax.iota(jnp.int32, 16)

    pltpu.sync_copy(idx2_hbm.at[pl.ds(wid * NCH, NCH)], idxall)
    pltpu.sync_copy(g_hbm.at[pl.ds(base, CPW)], qall)
    tbufs = [tbufA, tbufB, tbufC, tbufD]
    sems = [semA, semB, semC, semD]
    NB = 4
    for b in range(NB):
        pltpu.async_copy(g_hbm.at[idxall.at[b]], tbufs[b], sems[b])

    def compute_chunk(ch, tbuf):
        def i_body(i, c2):
            ii = ch * K + i
            qs = []
            for u in range(4):
                qw = plsc.bitcast(qall[ii, 16 * u:16 * (u + 1)],
                                  jnp.bfloat16)
                qa, qb = plsc.unpack(qw, format=plsc.PackFormat.INTERLEAVED)
                qs.append(qa)
                qs.append(qb)
            sim = jnp.zeros((16,), jnp.float32)
            for t in range(T):
                r = i * T + t
                ps = []
                for u in range(4):
                    tw = plsc.bitcast(tbuf[r, 16 * u:16 * (u + 1)],
                                      jnp.bfloat16)
                    a, b2 = plsc.unpack(tw,
                                        format=plsc.PackFormat.INTERLEAVED)
                    ps.append(qs[2 * u] * a)
                    ps.append(qs[2 * u + 1] * b2)
                acc = (((ps[0] + ps[1]) + (ps[2] + ps[3]))
                       + ((ps[4] + ps[5]) + (ps[6] + ps[7])))
                sim = jnp.where(lane == t, jnp.sum(acc), sim)
            simall[ii, :] = sim
            return c2

        lax.fori_loop(0, K, i_body, 0)

    def ring_body(j, carry):
        ch0 = NB * j
        for b in range(NB):
            ch = ch0 + b
            pltpu.make_async_copy(g_hbm.at[idxall.at[0]], tbufs[b],
                                  sems[b]).wait()
            compute_chunk(ch, tbufs[b])

            @pl.when(ch + NB < NCH)
            def _():
                pltpu.async_copy(g_hbm.at[idxall.at[ch + NB]], tbufs[b],
                                 sems[b])

        return carry

    lax.fori_loop(0, NCH // NB, ring_body, 0)
    pltpu.sync_copy(simall, out_hbm.at[pl.ds(base, CPW)])


@functools.cache
def _sc_sims():
    return pl.kernel(
        _sc_sims_body,
        out_type=jax.ShapeDtypeStruct((NP, 16), jnp.float32),
        mesh=plsc.VectorSubcoreMesh(core_axis_name="c", subcore_axis_name="s"),
        compiler_params=pltpu.CompilerParams(needs_layout_passes=False,
                                             use_tc_tiling_on_sc=False),
        scratch_types=[
            pltpu.VMEM((CPW, F // 2), jnp.int32),
            pltpu.VMEM((NCH, K * T), jnp.int32),
            pltpu.VMEM((K * T, F // 2), jnp.int32),
            pltpu.VMEM((K * T, F // 2), jnp.int32),
            pltpu.VMEM((K * T, F // 2), jnp.int32),
            pltpu.VMEM((K * T, F // 2), jnp.int32),
            pltpu.VMEM((CPW, 16), jnp.float32),
            pltpu.SemaphoreType.DMA,
            pltpu.SemaphoreType.DMA,
            pltpu.SemaphoreType.DMA,
            pltpu.SemaphoreType.DMA,
        ],
    )


def _loss_body(sim_ref, out_ref):
    s = sim_ref[...]
    lane = lax.broadcasted_iota(jnp.int32, (NP, 16), 1)
    e = jnp.where((lane >= 1) & (lane < T), jnp.exp(s), 0.0)
    den = jnp.sum(e, axis=1, keepdims=True)
    li = jnp.log(den) - s[:, 0:1]
    row = lax.broadcasted_iota(jnp.int32, (NP, 1), 0)
    li = jnp.where(row < N, li, 0.0)
    out_ref[...] = (jnp.sum(li) / N).reshape(1, 1)


def _loss(sims):
    return pl.pallas_call(
        _loss_body,
        out_shape=jax.ShapeDtypeStruct((1, 1), jnp.float32),
    )(sims)


def kernel(seq1, seq2, seq3, seq4, adj, aug_adj1edge, aug_adj2edge, sparse,
           msk, samp_bias1, samp_bias2, lbl, sample, W0, b0, gamma0, beta0):
    x = seq1[0]
    h = _gcn(x, W0, b0.reshape(1, F), adj)
    g = _bn_normalize(h, gamma0.reshape(1, F), beta0.reshape(1, F))
    idx2 = jnp.concatenate([
        sample.astype(jnp.int32).reshape(-1),
        jnp.zeros((NP - N) * T, jnp.int32),
    ]).reshape(NP * T // (K * T), K * T)
    g_i32 = lax.bitcast_convert_type(g.reshape(NP, F // 2, 2), jnp.int32)
    sims = _sc_sims()(g_i32, idx2)
    return _loss(sims)[0, 0]


# speedup vs baseline: 1.3408x; 1.0529x over previous
import functools

import jax
import jax.numpy as jnp
from jax import lax
from jax.experimental import pallas as pl
from jax.experimental.pallas import tpu as pltpu
import jax.experimental.pallas.tpu_sc as plsc

N = 10000
F = 128
T = 10
NP = 10240
NW = 32
CPW = NP // NW
K = 8
NCH = CPW // K
IPW = CPW * T
BI = 400


def _gcn_body(x_ref, w_ref, b_ref, adj_ref, h_ref, xw_scr):
    @pl.when(pl.program_id(0) == 0)
    def _():
        xw_scr[...] = jnp.dot(x_ref[...], w_ref[...],
                              preferred_element_type=jnp.float32)

    z = jnp.dot(adj_ref[...], xw_scr[...],
                preferred_element_type=jnp.float32) + b_ref[...]
    h_ref[...] = jnp.where(z > 0, z, jnp.exp(jnp.minimum(z, 0.0)) - 1.0)


def _gcn(x, w, b, adj):
    return pl.pallas_call(
        _gcn_body,
        grid=(N // BI,),
        in_specs=[
            pl.BlockSpec((N, F), lambda i: (0, 0)),
            pl.BlockSpec((F, F), lambda i: (0, 0)),
            pl.BlockSpec((1, F), lambda i: (0, 0)),
            pl.BlockSpec((BI, N), lambda i: (i, 0)),
        ],
        out_specs=pl.BlockSpec((BI, F), lambda i: (i, 0)),
        out_shape=jax.ShapeDtypeStruct((N, F), jnp.float32),
        scratch_shapes=[pltpu.VMEM((N, F), jnp.float32)],
    )(x, w, b, adj)


def _bn_body(h_ref, gam_ref, bet_ref, out_ref):
    h = h_ref[...]
    mean = jnp.mean(h, axis=0, keepdims=True)
    c = h - mean
    var = jnp.mean(c * c, axis=0, keepdims=True)
    y = c * lax.rsqrt(var + 1e-5) * gam_ref[...] + bet_ref[...]
    rn = jnp.sqrt(jnp.sum(y * y, axis=1, keepdims=True))
    g = (y / jnp.maximum(rn, 1e-8)).astype(jnp.bfloat16)
    lo = lax.bitcast_convert_type(g[:, 0:F // 2], jnp.uint16)
    hi = lax.bitcast_convert_type(g[:, F // 2:F], jnp.uint16)
    packed = (hi.astype(jnp.uint32) << 16) | lo.astype(jnp.uint32)
    out_ref[0:N, :] = lax.bitcast_convert_type(packed, jnp.int32)
    out_ref[N:NP, :] = jnp.zeros((NP - N, F // 2), jnp.int32)


def _bn_normalize(h, gam, bet):
    return pl.pallas_call(
        _bn_body,
        out_shape=jax.ShapeDtypeStruct((NP, F // 2), jnp.int32),
    )(h, gam, bet)


def _sc_sims_body(g_hbm, idx2_hbm, out_hbm, qall, idxall, tbufA, tbufB,
                  tbufC, tbufD, simall, semA, semB, semC, semD):
    wid = lax.axis_index("s") * 2 + lax.axis_index("c")
    base = wid * CPW
    lane = lax.iota(jnp.int32, 16)
    zero16i = jnp.zeros((16,), jnp.int32)
    TAIL = N * T - (NW - 1) * IPW

    @pl.when(wid < NW - 1)
    def _():
        pltpu.sync_copy(idx2_hbm.at[pl.ds(wid * IPW, IPW)], idxall)

    @pl.when(wid == NW - 1)
    def _():
        for z in range((IPW - TAIL) // 16):
            idxall[pl.ds(TAIL + 16 * z, 16)] = zero16i
        pltpu.sync_copy(idx2_hbm.at[pl.ds((NW - 1) * IPW, TAIL)],
                        idxall.at[pl.ds(0, TAIL)])

    pltpu.sync_copy(g_hbm.at[pl.ds(base, CPW)], qall)
    tbufs = [tbufA, tbufB, tbufC, tbufD]
    sems = [semA, semB, semC, semD]
    NB = 4
    for b in range(NB):
        pltpu.async_copy(g_hbm.at[idxall.at[pl.ds(b * K * T, K * T)]],
                         tbufs[b], sems[b])

    def compute_chunk(ch, tbuf):
        def i_body(i, c2):
            ii = ch * K + i
            qw = [plsc.bitcast(qall[ii, 16 * u:16 * (u + 1)], jnp.bfloat16)
                  for u in range(4)]
            sim = jnp.zeros((16,), jnp.float32)
            for t in range(T):
                r = i * T + t
                p0 = qw[0] * plsc.bitcast(tbuf[r, 0:16], jnp.bfloat16)
                p1 = qw[1] * plsc.bitcast(tbuf[r, 16:32], jnp.bfloat16)
                p2 = qw[2] * plsc.bitcast(tbuf[r, 32:48], jnp.bfloat16)
                p3 = qw[3] * plsc.bitcast(tbuf[r, 48:64], jnp.bfloat16)
                acc32 = (p0 + p1) + (p2 + p3)
                a, b2 = plsc.unpack(acc32,
                                    format=plsc.PackFormat.INTERLEAVED)
                sim = jnp.where(lane == t, jnp.sum(a + b2), sim)
            simall[ii, :] = sim
            return c2

        lax.fori_loop(0, K, i_body, 0)

    def ring_body(j, carry):
        ch0 = NB * j
        for b in range(NB):
            ch = ch0 + b
            pltpu.make_async_copy(g_hbm.at[idxall.at[pl.ds(0, K * T)]],
                                  tbufs[b], sems[b]).wait()
            compute_chunk(ch, tbufs[b])

            @pl.when(ch + NB < NCH)
            def _():
                pltpu.async_copy(
                    g_hbm.at[idxall.at[pl.ds((ch + NB) * (K * T), K * T)]],
                    tbufs[b], sems[b])

        return carry

    lax.fori_loop(0, NCH // NB, ring_body, 0)
    pltpu.sync_copy(simall, out_hbm.at[pl.ds(base, CPW)])


@functools.cache
def _sc_sims():
    return pl.kernel(
        _sc_sims_body,
        out_type=jax.ShapeDtypeStruct((NP, 16), jnp.float32),
        mesh=plsc.VectorSubcoreMesh(core_axis_name="c", subcore_axis_name="s"),
        compiler_params=pltpu.CompilerParams(needs_layout_passes=False,
                                             use_tc_tiling_on_sc=False),
        scratch_types=[
            pltpu.VMEM((CPW, F // 2), jnp.int32),
            pltpu.VMEM((IPW,), jnp.int32),
            pltpu.VMEM((K * T, F // 2), jnp.int32),
            pltpu.VMEM((K * T, F // 2), jnp.int32),
            pltpu.VMEM((K * T, F // 2), jnp.int32),
            pltpu.VMEM((K * T, F // 2), jnp.int32),
            pltpu.VMEM((CPW, 16), jnp.float32),
            pltpu.SemaphoreType.DMA,
            pltpu.SemaphoreType.DMA,
            pltpu.SemaphoreType.DMA,
            pltpu.SemaphoreType.DMA,
        ],
    )


def _loss_body(sim_ref, out_ref):
    s = sim_ref[...]
    lane = lax.broadcasted_iota(jnp.int32, (NP, 16), 1)
    e = jnp.where((lane >= 1) & (lane < T), jnp.exp(s), 0.0)
    den = jnp.sum(e, axis=1, keepdims=True)
    li = jnp.log(den) - s[:, 0:1]
    row = lax.broadcasted_iota(jnp.int32, (NP, 1), 0)
    li = jnp.where(row < N, li, 0.0)
    out_ref[...] = (jnp.sum(li) / N).reshape(1, 1)


def _loss(sims):
    return pl.pallas_call(
        _loss_body,
        out_shape=jax.ShapeDtypeStruct((1, 1), jnp.float32),
    )(sims)


def kernel(seq1, seq2, seq3, seq4, adj, aug_adj1edge, aug_adj2edge, sparse,
           msk, samp_bias1, samp_bias2, lbl, sample, W0, b0, gamma0, beta0):
    x = seq1[0]
    h = _gcn(x, W0, b0.reshape(1, F), adj)
    g_i32 = _bn_normalize(h, gamma0.reshape(1, F), beta0.reshape(1, F))
    idx_flat = sample.astype(jnp.int32).reshape(-1)
    sims = _sc_sims()(g_i32, idx_flat)
    return _loss(sims)[0, 0]


# speedup vs baseline: 1.3431x; 1.0017x over previous
import functools

import jax
import jax.numpy as jnp
from jax import lax
from jax.experimental import pallas as pl
from jax.experimental.pallas import tpu as pltpu
import jax.experimental.pallas.tpu_sc as plsc

N = 10000
F = 128
T = 10
NP = 10240
NW = 32
CPW = NP // NW
K = 8
NCH = CPW // K
IPW = CPW * T
BI = 400


def _gcn_body(x_ref, w_ref, b_ref, adj_ref, h_ref, xw_scr):
    @pl.when(pl.program_id(0) == 0)
    def _():
        xw_scr[...] = jnp.dot(x_ref[...], w_ref[...],
                              preferred_element_type=jnp.float32)

    z = jnp.dot(adj_ref[...].astype(jnp.bfloat16),
                xw_scr[...].astype(jnp.bfloat16),
                preferred_element_type=jnp.float32) + b_ref[...]
    h_ref[...] = jnp.where(z > 0, z, jnp.exp(jnp.minimum(z, 0.0)) - 1.0)


def _gcn(x, w, b, adj):
    return pl.pallas_call(
        _gcn_body,
        grid=(N // BI,),
        in_specs=[
            pl.BlockSpec((N, F), lambda i: (0, 0)),
            pl.BlockSpec((F, F), lambda i: (0, 0)),
            pl.BlockSpec((1, F), lambda i: (0, 0)),
            pl.BlockSpec((BI, N), lambda i: (i, 0)),
        ],
        out_specs=pl.BlockSpec((BI, F), lambda i: (i, 0)),
        out_shape=jax.ShapeDtypeStruct((N, F), jnp.float32),
        scratch_shapes=[pltpu.VMEM((N, F), jnp.float32)],
    )(x, w, b, adj)


def _bn_body(h_ref, gam_ref, bet_ref, out_ref):
    h = h_ref[...]
    mean = jnp.mean(h, axis=0, keepdims=True)
    c = h - mean
    var = jnp.mean(c * c, axis=0, keepdims=True)
    y = c * lax.rsqrt(var + 1e-5) * gam_ref[...] + bet_ref[...]
    rn = jnp.sqrt(jnp.sum(y * y, axis=1, keepdims=True))
    g = (y / jnp.maximum(rn, 1e-8)).astype(jnp.bfloat16)
    lo = lax.bitcast_convert_type(g[:, 0:F // 2], jnp.uint16)
    hi = lax.bitcast_convert_type(g[:, F // 2:F], jnp.uint16)
    packed = (hi.astype(jnp.uint32) << 16) | lo.astype(jnp.uint32)
    out_ref[0:N, :] = lax.bitcast_convert_type(packed, jnp.int32)
    out_ref[N:NP, :] = jnp.zeros((NP - N, F // 2), jnp.int32)


def _bn_normalize(h, gam, bet):
    return pl.pallas_call(
        _bn_body,
        out_shape=jax.ShapeDtypeStruct((NP, F // 2), jnp.int32),
    )(h, gam, bet)


def _sc_sims_body(g_hbm, idx2_hbm, out_hbm, qall, idxall, tbufA, tbufB,
                  tbufC, tbufD, simall, semA, semB, semC, semD):
    wid = lax.axis_index("s") * 2 + lax.axis_index("c")
    base = wid * CPW
    lane = lax.iota(jnp.int32, 16)
    zero16i = jnp.zeros((16,), jnp.int32)
    TAIL = N * T - (NW - 1) * IPW

    @pl.when(wid < NW - 1)
    def _():
        pltpu.sync_copy(idx2_hbm.at[pl.ds(wid * IPW, IPW)], idxall)

    @pl.when(wid == NW - 1)
    def _():
        for z in range((IPW - TAIL) // 16):
            idxall[pl.ds(TAIL + 16 * z, 16)] = zero16i
        pltpu.sync_copy(idx2_hbm.at[pl.ds((NW - 1) * IPW, TAIL)],
                        idxall.at[pl.ds(0, TAIL)])

    pltpu.sync_copy(g_hbm.at[pl.ds(base, CPW)], qall)
    tbufs = [tbufA, tbufB, tbufC, tbufD]
    sems = [semA, semB, semC, semD]
    NB = 4
    for b in range(NB):
        pltpu.async_copy(g_hbm.at[idxall.at[pl.ds(b * K * T, K * T)]],
                         tbufs[b], sems[b])

    def compute_chunk(ch, tbuf):
        def i_body(i, c2):
            ii = ch * K + i
            qw = [plsc.bitcast(qall[ii, 16 * u:16 * (u + 1)], jnp.bfloat16)
                  for u in range(4)]
            sim = jnp.zeros((16,), jnp.float32)
            for t in range(T):
                r = i * T + t
                p0 = qw[0] * plsc.bitcast(tbuf[r, 0:16], jnp.bfloat16)
                p1 = qw[1] * plsc.bitcast(tbuf[r, 16:32], jnp.bfloat16)
                p2 = qw[2] * plsc.bitcast(tbuf[r, 32:48], jnp.bfloat16)
                p3 = qw[3] * plsc.bitcast(tbuf[r, 48:64], jnp.bfloat16)
                acc32 = (p0 + p1) + (p2 + p3)
                a, b2 = plsc.unpack(acc32,
                                    format=plsc.PackFormat.INTERLEAVED)
                sim = jnp.where(lane == t, jnp.sum(a + b2), sim)
            simall[ii, :] = sim
            return c2

        lax.fori_loop(0, K, i_body, 0)

    def ring_body(j, carry):
        ch0 = NB * j
        for b in range(NB):
            ch = ch0 + b
            pltpu.make_async_copy(g_hbm.at[idxall.at[pl.ds(0, K * T)]],
                                  tbufs[b], sems[b]).wait()
            compute_chunk(ch, tbufs[b])

            @pl.when(ch + NB < NCH)
            def _():
                pltpu.async_copy(
                    g_hbm.at[idxall.at[pl.ds((ch + NB) * (K * T), K * T)]],
                    tbufs[b], sems[b])

        return carry

    lax.fori_loop(0, NCH // NB, ring_body, 0)
    pltpu.sync_copy(simall, out_hbm.at[pl.ds(base, CPW)])


@functools.cache
def _sc_sims():
    return pl.kernel(
        _sc_sims_body,
        out_type=jax.ShapeDtypeStruct((NP, 16), jnp.float32),
        mesh=plsc.VectorSubcoreMesh(core_axis_name="c", subcore_axis_name="s"),
        compiler_params=pltpu.CompilerParams(needs_layout_passes=False,
                                             use_tc_tiling_on_sc=False),
        scratch_types=[
            pltpu.VMEM((CPW, F // 2), jnp.int32),
            pltpu.VMEM((IPW,), jnp.int32),
            pltpu.VMEM((K * T, F // 2), jnp.int32),
            pltpu.VMEM((K * T, F // 2), jnp.int32),
            pltpu.VMEM((K * T, F // 2), jnp.int32),
            pltpu.VMEM((K * T, F // 2), jnp.int32),
            pltpu.VMEM((CPW, 16), jnp.float32),
            pltpu.SemaphoreType.DMA,
            pltpu.SemaphoreType.DMA,
            pltpu.SemaphoreType.DMA,
            pltpu.SemaphoreType.DMA,
        ],
    )


def _loss_body(sim_ref, out_ref):
    s = sim_ref[...]
    lane = lax.broadcasted_iota(jnp.int32, (NP, 16), 1)
    e = jnp.where((lane >= 1) & (lane < T), jnp.exp(s), 0.0)
    den = jnp.sum(e, axis=1, keepdims=True)
    li = jnp.log(den) - s[:, 0:1]
    row = lax.broadcasted_iota(jnp.int32, (NP, 1), 0)
    li = jnp.where(row < N, li, 0.0)
    out_ref[...] = (jnp.sum(li) / N).reshape(1, 1)


def _loss(sims):
    return pl.pallas_call(
        _loss_body,
        out_shape=jax.ShapeDtypeStruct((1, 1), jnp.float32),
    )(sims)


def kernel(seq1, seq2, seq3, seq4, adj, aug_adj1edge, aug_adj2edge, sparse,
           msk, samp_bias1, samp_bias2, lbl, sample, W0, b0, gamma0, beta0):
    x = seq1[0]
    h = _gcn(x, W0, b0.reshape(1, F), adj)
    g_i32 = _bn_normalize(h, gamma0.reshape(1, F), beta0.reshape(1, F))
    idx_flat = sample.astype(jnp.int32).reshape(-1)
    sims = _sc_sims()(g_i32, idx_flat)
    return _loss(sims)[0, 0]


# speedup vs baseline: 1.8150x; 1.3513x over previous
import functools

import jax
import jax.numpy as jnp
from jax import lax
from jax.experimental import pallas as pl
from jax.experimental.pallas import tpu as pltpu
import jax.experimental.pallas.tpu_sc as plsc

N = 10000
F = 128
T = 10
NP = 10240
NW = 32
CPW = NP // NW
K = 8
NCH = CPW // K
IPW = CPW * T
BI = 400


def _gcn_body(x_ref, w_ref, b_ref, adj_ref, h_ref, xw_scr):
    @pl.when(pl.program_id(0) == 0)
    def _():
        xw_scr[...] = jnp.dot(x_ref[...], w_ref[...],
                              preferred_element_type=jnp.float32)

    z = jnp.dot(adj_ref[...], xw_scr[...],
                preferred_element_type=jnp.float32) + b_ref[...]
    h_ref[...] = jnp.where(z > 0, z, jnp.exp(jnp.minimum(z, 0.0)) - 1.0)


def _gcn(x, w, b, adj):
    return pl.pallas_call(
        _gcn_body,
        grid=(N // BI,),
        in_specs=[
            pl.BlockSpec((N, F), lambda i: (0, 0)),
            pl.BlockSpec((F, F), lambda i: (0, 0)),
            pl.BlockSpec((1, F), lambda i: (0, 0)),
            pl.BlockSpec((BI, N), lambda i: (i, 0)),
        ],
        out_specs=pl.BlockSpec((BI, F), lambda i: (i, 0)),
        out_shape=jax.ShapeDtypeStruct((N, F), jnp.float32),
        scratch_shapes=[pltpu.VMEM((N, F), jnp.float32)],
    )(x, w, b, adj)


def _bn_body(h_ref, gam_ref, bet_ref, out_ref):
    h = h_ref[...]
    mean = jnp.mean(h, axis=0, keepdims=True)
    c = h - mean
    var = jnp.mean(c * c, axis=0, keepdims=True)
    y = c * lax.rsqrt(var + 1e-5) * gam_ref[...] + bet_ref[...]
    rn = jnp.sqrt(jnp.sum(y * y, axis=1, keepdims=True))
    g = (y / jnp.maximum(rn, 1e-8)).astype(jnp.bfloat16)
    lo = lax.bitcast_convert_type(g[:, 0:F // 2], jnp.uint16)
    hi = lax.bitcast_convert_type(g[:, F // 2:F], jnp.uint16)
    packed = (hi.astype(jnp.uint32) << 16) | lo.astype(jnp.uint32)
    out_ref[0:N, :] = lax.bitcast_convert_type(packed, jnp.int32)
    out_ref[N:NP, :] = jnp.zeros((NP - N, F // 2), jnp.int32)


def _bn_normalize(h, gam, bet):
    return pl.pallas_call(
        _bn_body,
        out_shape=jax.ShapeDtypeStruct((NP, F // 2), jnp.int32),
    )(h, gam, bet)


def _sc_sims_body(g_hbm, idx2_hbm, out_hbm, qall, idxall, tbufA, tbufB,
                  tbufC, tbufD, simall, g_sh, semA, semB, semC, semD):
    wid = lax.axis_index("s") * 2 + lax.axis_index("c")
    base = wid * CPW
    lane = lax.iota(jnp.int32, 16)
    zero16i = jnp.zeros((16,), jnp.int32)
    TAIL = N * T - (NW - 1) * IPW

    @pl.when(wid < NW - 1)
    def _():
        pltpu.sync_copy(idx2_hbm.at[pl.ds(wid * IPW, IPW)], idxall)

    @pl.when(wid == NW - 1)
    def _():
        for z in range((IPW - TAIL) // 16):
            idxall[pl.ds(TAIL + 16 * z, 16)] = zero16i
        pltpu.sync_copy(idx2_hbm.at[pl.ds((NW - 1) * IPW, TAIL)],
                        idxall.at[pl.ds(0, TAIL)])

    sid = lax.axis_index("s")
    RPT = NP // 16
    pltpu.sync_copy(g_hbm.at[pl.ds(sid * RPT, RPT)],
                    g_sh.at[pl.ds(sid * RPT, RPT)])
    pltpu.sync_copy(g_hbm.at[pl.ds(base, CPW)], qall)
    plsc.subcore_barrier()
    tbufs = [tbufA, tbufB, tbufC, tbufD]
    sems = [semA, semB, semC, semD]
    NB = 4
    for b in range(NB):
        pltpu.async_copy(g_sh.at[idxall.at[pl.ds(b * K * T, K * T)]],
                         tbufs[b], sems[b])

    def compute_chunk(ch, tbuf):
        def i_body(i, c2):
            ii = ch * K + i
            qw = [plsc.bitcast(qall[ii, 16 * u:16 * (u + 1)], jnp.bfloat16)
                  for u in range(4)]
            sim = jnp.zeros((16,), jnp.float32)
            for t in range(T):
                r = i * T + t
                p0 = qw[0] * plsc.bitcast(tbuf[r, 0:16], jnp.bfloat16)
                p1 = qw[1] * plsc.bitcast(tbuf[r, 16:32], jnp.bfloat16)
                p2 = qw[2] * plsc.bitcast(tbuf[r, 32:48], jnp.bfloat16)
                p3 = qw[3] * plsc.bitcast(tbuf[r, 48:64], jnp.bfloat16)
                acc32 = (p0 + p1) + (p2 + p3)
                a, b2 = plsc.unpack(acc32,
                                    format=plsc.PackFormat.INTERLEAVED)
                sim = jnp.where(lane == t, jnp.sum(a + b2), sim)
            simall[ii, :] = sim
            return c2

        lax.fori_loop(0, K, i_body, 0)

    def ring_body(j, carry):
        ch0 = NB * j
        for b in range(NB):
            ch = ch0 + b
            pltpu.make_async_copy(g_sh.at[idxall.at[pl.ds(0, K * T)]],
                                  tbufs[b], sems[b]).wait()
            compute_chunk(ch, tbufs[b])

            @pl.when(ch + NB < NCH)
            def _():
                pltpu.async_copy(
                    g_sh.at[idxall.at[pl.ds((ch + NB) * (K * T), K * T)]],
                    tbufs[b], sems[b])

        return carry

    lax.fori_loop(0, NCH // NB, ring_body, 0)
    pltpu.sync_copy(simall, out_hbm.at[pl.ds(base, CPW)])


@functools.cache
def _sc_sims():
    return pl.kernel(
        _sc_sims_body,
        out_type=jax.ShapeDtypeStruct((NP, 16), jnp.float32),
        mesh=plsc.VectorSubcoreMesh(core_axis_name="c", subcore_axis_name="s"),
        compiler_params=pltpu.CompilerParams(needs_layout_passes=False,
                                             use_tc_tiling_on_sc=False),
        scratch_types=[
            pltpu.VMEM((CPW, F // 2), jnp.int32),
            pltpu.VMEM((IPW,), jnp.int32),
            pltpu.VMEM((K * T, F // 2), jnp.int32),
            pltpu.VMEM((K * T, F // 2), jnp.int32),
            pltpu.VMEM((K * T, F // 2), jnp.int32),
            pltpu.VMEM((K * T, F // 2), jnp.int32),
            pltpu.VMEM((CPW, 16), jnp.float32),
            pltpu.VMEM_SHARED((NP, F // 2), jnp.int32),
            pltpu.SemaphoreType.DMA,
            pltpu.SemaphoreType.DMA,
            pltpu.SemaphoreType.DMA,
            pltpu.SemaphoreType.DMA,
        ],
    )


def _loss_body(sim_ref, out_ref):
    s = sim_ref[...]
    lane = lax.broadcasted_iota(jnp.int32, (NP, 16), 1)
    e = jnp.where((lane >= 1) & (lane < T), jnp.exp(s), 0.0)
    den = jnp.sum(e, axis=1, keepdims=True)
    li = jnp.log(den) - s[:, 0:1]
    row = lax.broadcasted_iota(jnp.int32, (NP, 1), 0)
    li = jnp.where(row < N, li, 0.0)
    out_ref[...] = (jnp.sum(li) / N).reshape(1, 1)


def _loss(sims):
    return pl.pallas_call(
        _loss_body,
        out_shape=jax.ShapeDtypeStruct((1, 1), jnp.float32),
    )(sims)


def kernel(seq1, seq2, seq3, seq4, adj, aug_adj1edge, aug_adj2edge, sparse,
           msk, samp_bias1, samp_bias2, lbl, sample, W0, b0, gamma0, beta0):
    x = seq1[0]
    h = _gcn(x, W0, b0.reshape(1, F), adj)
    g_i32 = _bn_normalize(h, gamma0.reshape(1, F), beta0.reshape(1, F))
    idx_flat = sample.astype(jnp.int32).reshape(-1)
    sims = _sc_sims()(g_i32, idx_flat)
    return _loss(sims)[0, 0]


# speedup vs baseline: 1.8675x; 1.0289x over previous
import functools

import jax
import jax.numpy as jnp
from jax import lax
from jax.experimental import pallas as pl
from jax.experimental.pallas import tpu as pltpu
import jax.experimental.pallas.tpu_sc as plsc

N = 10000
F = 128
T = 10
NP = 10240
NW = 32
CPW = NP // NW
K = 8
NCH = CPW // K
IPW = CPW * T
BI = 200


def _gcn_body(x_ref, w_ref, b_ref, gam_ref, bet_ref, adj_ref, out_ref,
              xw_scr, h_scr, s1_scr, s2_scr):
    i = pl.program_id(0)

    @pl.when(i == 0)
    def _():
        xw_scr[...] = jnp.dot(x_ref[...], w_ref[...],
                              preferred_element_type=jnp.float32)

    z = jnp.dot(adj_ref[...], xw_scr[...],
                preferred_element_type=jnp.float32) + b_ref[...]
    hblk = jnp.where(z > 0, z, jnp.exp(jnp.minimum(z, 0.0)) - 1.0)
    h_scr[pl.ds(i * BI, BI), :] = hblk
    cs = jnp.sum(hblk, axis=0, keepdims=True)
    cs2 = jnp.sum(hblk * hblk, axis=0, keepdims=True)

    @pl.when(i == 0)
    def _():
        s1_scr[...] = cs
        s2_scr[...] = cs2

    @pl.when(i > 0)
    def _():
        s1_scr[...] += cs
        s2_scr[...] += cs2

    @pl.when(i == N // BI - 1)
    def _():
        mean = s1_scr[...] * (1.0 / N)
        var = s2_scr[...] * (1.0 / N) - mean * mean
        y = ((h_scr[...] - mean) * lax.rsqrt(var + 1e-5) * gam_ref[...]
             + bet_ref[...])
        rn = jnp.sqrt(jnp.sum(y * y, axis=1, keepdims=True))
        g = (y / jnp.maximum(rn, 1e-8)).astype(jnp.bfloat16)
        lo = lax.bitcast_convert_type(g[:, 0:F // 2], jnp.uint16)
        hi = lax.bitcast_convert_type(g[:, F // 2:F], jnp.uint16)
        packed = (hi.astype(jnp.uint32) << 16) | lo.astype(jnp.uint32)
        out_ref[0:N, :] = lax.bitcast_convert_type(packed, jnp.int32)
        out_ref[N:NP, :] = jnp.zeros((NP - N, F // 2), jnp.int32)


def _gcn(x, w, b, gam, bet, adj):
    return pl.pallas_call(
        _gcn_body,
        grid=(N // BI,),
        in_specs=[
            pl.BlockSpec((N, F), lambda i: (0, 0)),
            pl.BlockSpec((F, F), lambda i: (0, 0)),
            pl.BlockSpec((1, F), lambda i: (0, 0)),
            pl.BlockSpec((1, F), lambda i: (0, 0)),
            pl.BlockSpec((1, F), lambda i: (0, 0)),
            pl.BlockSpec((BI, N), lambda i: (i, 0)),
        ],
        out_specs=pl.BlockSpec((NP, F // 2), lambda i: (0, 0)),
        out_shape=jax.ShapeDtypeStruct((NP, F // 2), jnp.int32),
        scratch_shapes=[
            pltpu.VMEM((N, F), jnp.float32),
            pltpu.VMEM((N, F), jnp.float32),
            pltpu.VMEM((1, F), jnp.float32),
            pltpu.VMEM((1, F), jnp.float32),
        ],
    )(x, w, b, gam, bet, adj)


def _sc_sims_body(g_hbm, idx2_hbm, out_hbm, qall, idxall, tbufA, tbufB,
                  tbufC, tbufD, simall, g_sh, semA, semB, semC, semD):
    wid = lax.axis_index("s") * 2 + lax.axis_index("c")
    base = wid * CPW
    lane = lax.iota(jnp.int32, 16)
    zero16i = jnp.zeros((16,), jnp.int32)
    TAIL = N * T - (NW - 1) * IPW

    @pl.when(wid < NW - 1)
    def _():
        pltpu.sync_copy(idx2_hbm.at[pl.ds(wid * IPW, IPW)], idxall)

    @pl.when(wid == NW - 1)
    def _():
        for z in range((IPW - TAIL) // 16):
            idxall[pl.ds(TAIL + 16 * z, 16)] = zero16i
        pltpu.sync_copy(idx2_hbm.at[pl.ds((NW - 1) * IPW, TAIL)],
                        idxall.at[pl.ds(0, TAIL)])

    sid = lax.axis_index("s")
    RPT = NP // 16
    pltpu.sync_copy(g_hbm.at[pl.ds(sid * RPT, RPT)],
                    g_sh.at[pl.ds(sid * RPT, RPT)])
    pltpu.sync_copy(g_hbm.at[pl.ds(base, CPW)], qall)
    plsc.subcore_barrier()
    tbufs = [tbufA, tbufB, tbufC, tbufD]
    sems = [semA, semB, semC, semD]
    NB = 4
    for b in range(NB):
        pltpu.async_copy(g_sh.at[idxall.at[pl.ds(b * K * T, K * T)]],
                         tbufs[b], sems[b])

    def compute_chunk(ch, tbuf):
        def i_body(i, c2):
            ii = ch * K + i
            qw = [plsc.bitcast(qall[ii, 16 * u:16 * (u + 1)], jnp.bfloat16)
                  for u in range(4)]
            sim = jnp.zeros((16,), jnp.float32)
            for t in range(T):
                r = i * T + t
                p0 = qw[0] * plsc.bitcast(tbuf[r, 0:16], jnp.bfloat16)
                p1 = qw[1] * plsc.bitcast(tbuf[r, 16:32], jnp.bfloat16)
                p2 = qw[2] * plsc.bitcast(tbuf[r, 32:48], jnp.bfloat16)
                p3 = qw[3] * plsc.bitcast(tbuf[r, 48:64], jnp.bfloat16)
                acc32 = (p0 + p1) + (p2 + p3)
                a, b2 = plsc.unpack(acc32,
                                    format=plsc.PackFormat.INTERLEAVED)
                sim = jnp.where(lane == t, jnp.sum(a + b2), sim)
            simall[ii, :] = sim
            return c2

        lax.fori_loop(0, K, i_body, 0)

    def ring_body(j, carry):
        ch0 = NB * j
        for b in range(NB):
            ch = ch0 + b
            pltpu.make_async_copy(g_sh.at[idxall.at[pl.ds(0, K * T)]],
                                  tbufs[b], sems[b]).wait()
            compute_chunk(ch, tbufs[b])

            @pl.when(ch + NB < NCH)
            def _():
                pltpu.async_copy(
                    g_sh.at[idxall.at[pl.ds((ch + NB) * (K * T), K * T)]],
                    tbufs[b], sems[b])

        return carry

    lax.fori_loop(0, NCH // NB, ring_body, 0)
    pltpu.sync_copy(simall, out_hbm.at[pl.ds(base, CPW)])


@functools.cache
def _sc_sims():
    return pl.kernel(
        _sc_sims_body,
        out_type=jax.ShapeDtypeStruct((NP, 16), jnp.float32),
        mesh=plsc.VectorSubcoreMesh(core_axis_name="c", subcore_axis_name="s"),
        compiler_params=pltpu.CompilerParams(needs_layout_passes=False,
                                             use_tc_tiling_on_sc=False),
        scratch_types=[
            pltpu.VMEM((CPW, F // 2), jnp.int32),
            pltpu.VMEM((IPW,), jnp.int32),
            pltpu.VMEM((K * T, F // 2), jnp.int32),
            pltpu.VMEM((K * T, F // 2), jnp.int32),
            pltpu.VMEM((K * T, F // 2), jnp.int32),
            pltpu.VMEM((K * T, F // 2), jnp.int32),
            pltpu.VMEM((CPW, 16), jnp.float32),
            pltpu.VMEM_SHARED((NP, F // 2), jnp.int32),
            pltpu.SemaphoreType.DMA,
            pltpu.SemaphoreType.DMA,
            pltpu.SemaphoreType.DMA,
            pltpu.SemaphoreType.DMA,
        ],
    )


def _loss_body(sim_ref, out_ref):
    s = sim_ref[...]
    lane = lax.broadcasted_iota(jnp.int32, (NP, 16), 1)
    e = jnp.where((lane >= 1) & (lane < T), jnp.exp(s), 0.0)
    den = jnp.sum(e, axis=1, keepdims=True)
    li = jnp.log(den) - s[:, 0:1]
    row = lax.broadcasted_iota(jnp.int32, (NP, 1), 0)
    li = jnp.where(row < N, li, 0.0)
    out_ref[...] = (jnp.sum(li) / N).reshape(1, 1)


def _loss(sims):
    return pl.pallas_call(
        _loss_body,
        out_shape=jax.ShapeDtypeStruct((1, 1), jnp.float32),
    )(sims)


def kernel(seq1, seq2, seq3, seq4, adj, aug_adj1edge, aug_adj2edge, sparse,
           msk, samp_bias1, samp_bias2, lbl, sample, W0, b0, gamma0, beta0):
    g_i32 = _gcn(seq1[0], W0, b0.reshape(1, F), gamma0.reshape(1, F),
                 beta0.reshape(1, F), adj)
    idx_flat = sample.astype(jnp.int32).reshape(-1)
    sims = _sc_sims()(g_i32, idx_flat)
    return _loss(sims)[0, 0]
